# Initial kernel scaffold; baseline (speedup 1.0000x reference)
import jax
import jax.numpy as jnp
from jax.experimental import pallas as pl


def kernel(x, edge_index, W_in, b_in, W1, b1, W2, b2, W3, b3, W_out, b_out):
    raise NotImplementedError("write your pallas kernel here")


# speedup vs baseline: 4.0323x; 4.0323x over previous
import functools

import jax
import jax.numpy as jnp
from jax import lax
from jax.experimental import pallas as pl
from jax.experimental.pallas import tpu as pltpu
from jax.experimental.pallas import tpu_sc as plsc

N = 10000
D = 128
E = 320000

_info = plsc.get_sparse_core_info()
_NC, _NS, _L = _info.num_cores, _info.num_subcores, _info.num_lanes
_NW = _NC * _NS
_CHUNK = 128
_T = -(-E // (_NW * _CHUNK))
_E_PAD = _NW * _CHUNK * _T
_ROWS_PER_TILE = 632
_ACC_ROWS = _NS * _ROWS_PER_TILE


def _mp_sc(h, col2d, row2d):
    mesh = plsc.VectorSubcoreMesh(core_axis_name="c", subcore_axis_name="s")

    @functools.partial(
        pl.kernel,
        mesh=mesh,
        out_type=jax.ShapeDtypeStruct((_NC, _ACC_ROWS, D), jnp.float32),
        scratch_types=[
            pltpu.VMEM((1, _CHUNK), jnp.int32),
            pltpu.VMEM((1, _CHUNK), jnp.int32),
            pltpu.VMEM((_CHUNK, D), jnp.float32),
            pltpu.VMEM_SHARED((_ACC_ROWS, D), jnp.float32),
            pltpu.SemaphoreType.DMA,
        ],
    )
    def k(h_hbm, col_hbm, row_hbm, out_hbm, cidx, ridx, rows, acc, sem):
        cid = lax.axis_index("c")
        sid = lax.axis_index("s")
        wid = sid * _NC + cid

        zeros16 = jnp.zeros((_L,), jnp.float32)

        def zrow(i, carry):
            for j in range(D // _L):
                rows[i, pl.ds(j * _L, _L)] = zeros16
            return carry

        lax.fori_loop(0, _CHUNK, zrow, 0)
        z0 = sid * _ROWS_PER_TILE
        for t in range(0, _ROWS_PER_TILE, _CHUNK):
            sz = min(_CHUNK, _ROWS_PER_TILE - t)
            pltpu.sync_copy(rows.at[pl.ds(0, sz)], acc.at[pl.ds(z0 + t, sz)])
        plsc.subcore_barrier()

        base = wid * _T

        def body(j, carry):
            pltpu.sync_copy(col_hbm.at[base + j], cidx.at[0])
            pltpu.async_copy(h_hbm.at[cidx.at[0]], rows, sem).wait()
            pltpu.sync_copy(row_hbm.at[base + j], ridx.at[0])
            pltpu.sync_copy(rows, acc.at[ridx.at[0]], add=True)
            return carry

        lax.fori_loop(0, _T, body, 0)
        plsc.subcore_barrier()

        pltpu.sync_copy(
            acc.at[pl.ds(z0, _ROWS_PER_TILE)],
            out_hbm.at[cid, pl.ds(z0, _ROWS_PER_TILE)],
        )

    return k(h, col2d, row2d)


_R = 2000


def _dense_tc(x, wt, b, act):
    def body(x_ref, w_ref, b_ref, o_ref):
        y = jnp.dot(x_ref[...], w_ref[...],
                    preferred_element_type=jnp.float32) + b_ref[...]
        o_ref[...] = jnp.tanh(y) if act else y

    return pl.pallas_call(
        body,
        grid=(x.shape[0] // _R,),
        in_specs=[
            pl.BlockSpec((_R, D), lambda i: (i, 0)),
            pl.BlockSpec((D, D), lambda i: (0, 0)),
            pl.BlockSpec((1, D), lambda i: (0, 0)),
        ],
        out_specs=pl.BlockSpec((_R, D), lambda i: (i, 0)),
        out_shape=jax.ShapeDtypeStruct((x.shape[0], D), jnp.float32),
    )(x, wt, b.reshape(1, D))


def _dense2_tc(parts, wt, b, act):
    def body(p_ref, w_ref, b_ref, o_ref):
        s = p_ref[0] + p_ref[1]
        y = jnp.dot(s, w_ref[...],
                    preferred_element_type=jnp.float32) + b_ref[...]
        o_ref[...] = jnp.tanh(y) if act else y

    return pl.pallas_call(
        body,
        grid=(N // _R,),
        in_specs=[
            pl.BlockSpec((2, _R, D), lambda i: (0, i, 0)),
            pl.BlockSpec((D, D), lambda i: (0, 0)),
            pl.BlockSpec((1, D), lambda i: (0, 0)),
        ],
        out_specs=pl.BlockSpec((_R, D), lambda i: (i, 0)),
        out_shape=jax.ShapeDtypeStruct((N, D), jnp.float32),
    )(parts, wt, b.reshape(1, D))


def kernel(x, edge_index, W_in, b_in, W1, b1, W2, b2, W3, b3, W_out, b_out):
    row = edge_index[0]
    col = edge_index[1]
    pad = _E_PAD - E
    colp = jnp.concatenate(
        [col, jnp.zeros((pad,), jnp.int32)]).reshape(_E_PAD // _CHUNK, _CHUNK)
    rowp = jnp.concatenate(
        [row, jnp.full((pad,), N, jnp.int32)]).reshape(_E_PAD // _CHUNK, _CHUNK)

    h = _dense_tc(x, W_in.T, b_in, True)
    for W, b in ((W1, b1), (W2, b2), (W3, b3)):
        parts = _mp_sc(h, colp, rowp)
        h = _dense2_tc(parts, W.T, b, True)
    return _dense_tc(h, W_out.T, b_out, False)
